# Initial kernel scaffold; baseline (speedup 1.0000x reference)
import jax
import jax.numpy as jnp
from jax.experimental import pallas as pl


def kernel(x, h_parent, row_embed, col_embed, val_embed, head_w, head_b):
    raise NotImplementedError("write your pallas kernel here")


# speedup vs baseline: 309.9804x; 309.9804x over previous
import functools

import jax
import jax.numpy as jnp
from jax.experimental import pallas as pl
from jax.experimental.pallas import tpu as pltpu

_B, _T, _H, _W = 1024, 8, 16, 16
_J = _T * _H * _W
_NE = 64
_VOCAB = 10
_BB = 128


def _body(x_ref, hp_ref, row_ref, col_ref, val_ref, w_ref, b_ref, out_ref):
    xb = x_ref[...]
    maskf = (xb != 0).astype(jnp.float32)

    j = jax.lax.broadcasted_iota(jnp.int32, (_J, 32), 0)
    k = jax.lax.broadcasted_iota(jnp.int32, (_J, 32), 1)
    rsel = (((j // _W) % _H) == k).astype(jnp.float32)
    csel = ((j % _W) == (k - _H)).astype(jnp.float32)
    ltk = (k < _H).astype(jnp.float32)
    m_pos = rsel * ltk + csel * (1.0 - ltk)
    poscnt = jnp.dot(maskf, m_pos, preferred_element_type=jnp.float32)

    num = jnp.dot(poscnt[:, :_H], row_ref[...],
                  preferred_element_type=jnp.float32)
    num = num + jnp.dot(poscnt[:, _H:], col_ref[...],
                        preferred_element_type=jnp.float32)

    for v in range(1, _VOCAB):
        cnt_v = jnp.sum((xb == v).astype(jnp.float32), axis=1, keepdims=True)
        num = num + cnt_v * val_ref[v, :][None, :]

    den = jnp.maximum(jnp.sum(maskf, axis=1, keepdims=True), 1.0)
    h_matrix = num / den

    dn = (((1,), (1,)), ((), ()))
    out = jax.lax.dot_general(h_matrix, w_ref[:, :_NE], dn,
                              preferred_element_type=jnp.float32)
    out = out + jax.lax.dot_general(hp_ref[...], w_ref[:, _NE:], dn,
                                    preferred_element_type=jnp.float32)
    out_ref[...] = out + b_ref[...]


@jax.jit
def kernel(x, h_parent, row_embed, col_embed, val_embed, head_w, head_b):
    b = x.shape[0]
    x2 = x.reshape(b, _J)
    grid = (b // _BB,)
    full = lambda i: (0, 0)
    out = pl.pallas_call(
        _body,
        grid=grid,
        in_specs=[
            pl.BlockSpec((_BB, _J), lambda i: (i, 0)),
            pl.BlockSpec((_BB, _NE), lambda i: (i, 0)),
            pl.BlockSpec(row_embed.shape, full),
            pl.BlockSpec(col_embed.shape, full),
            pl.BlockSpec(val_embed.shape, full),
            pl.BlockSpec(head_w.shape, full),
            pl.BlockSpec((1, head_w.shape[0]), full),
        ],
        out_specs=pl.BlockSpec((_BB, head_w.shape[0]), lambda i: (i, 0)),
        out_shape=jax.ShapeDtypeStruct((b, head_w.shape[0]), jnp.float32),
    )(x2, h_parent, row_embed, col_embed, val_embed, head_w,
      head_b.reshape(1, -1))
    return out
